# baseline (speedup 1.0000x reference)
import functools

import jax
import jax.numpy as jnp
from jax import lax
from jax.experimental import pallas as pl
from jax.experimental.pallas import tpu as pltpu
from jax.experimental.pallas import tpu_sc as plsc

_B = 4
_L = 8192
_N = _B * _L
_NC = 2
_NS = 16
_NW = _NC * _NS
_CHUNK = _N // _NW
_WPR = _L // _CHUNK
_LANES = 16

_mesh = plsc.VectorSubcoreMesh(core_axis_name="c", subcore_axis_name="s")


@functools.partial(
    pl.kernel,
    mesh=_mesh,
    out_type=jax.ShapeDtypeStruct((_N,), jnp.float32),
    scratch_types=[
        pltpu.VMEM((_CHUNK,), jnp.float32),
        pltpu.VMEM((_CHUNK,), jnp.float32),
        pltpu.SemaphoreType.DMA,
        pltpu.SemaphoreType.DMA,
    ],
)
def _pos_add_sc(x_hbm, pos_hbm, out_hbm, x_v, pos_v, sem_x, sem_p):
    wid = lax.axis_index("s") * _NC + lax.axis_index("c")
    base = wid * _CHUNK
    pos_base = (wid % _WPR) * _CHUNK
    cp_x = pltpu.async_copy(x_hbm.at[pl.ds(base, _CHUNK)], x_v, sem_x)
    cp_p = pltpu.async_copy(pos_hbm.at[pl.ds(pos_base, _CHUNK)], pos_v, sem_p)
    cp_x.wait()
    cp_p.wait()
    for i in range(_CHUNK // _LANES):
        sl = pl.ds(i * _LANES, _LANES)
        x_v[sl] = x_v[sl] + pos_v[sl]
    pltpu.sync_copy(x_v, out_hbm.at[pl.ds(base, _CHUNK)])


def kernel(x, pos_table):
    pos = pos_table.reshape(-1)[: x.shape[1]]
    out = _pos_add_sc(x.reshape(-1), pos)
    return out.reshape(x.shape)


# speedup vs baseline: 1.0748x; 1.0748x over previous
import functools

import jax
import jax.numpy as jnp
from jax import lax
from jax.experimental import pallas as pl
from jax.experimental.pallas import tpu as pltpu
from jax.experimental.pallas import tpu_sc as plsc

_B = 4
_L = 8192
_NC = 2
_NS = 16
_NW = _NC * _NS
_CHUNK = _B * _L // _NW
_WPR = _L // _CHUNK
_LANES = 16

_mesh = plsc.VectorSubcoreMesh(core_axis_name="c", subcore_axis_name="s")


@functools.partial(
    pl.kernel,
    mesh=_mesh,
    out_type=jax.ShapeDtypeStruct((_B, _L), jnp.float32),
    scratch_types=[
        pltpu.VMEM((_CHUNK,), jnp.float32),
        pltpu.VMEM((_CHUNK,), jnp.float32),
        pltpu.SemaphoreType.DMA,
        pltpu.SemaphoreType.DMA,
    ],
)
def _pos_add_sc(x_hbm, pos_hbm, out_hbm, x_v, pos_v, sem_x, sem_p):
    wid = lax.axis_index("s") * _NC + lax.axis_index("c")
    row = wid // _WPR
    col = (wid % _WPR) * _CHUNK
    cp_x = pltpu.async_copy(x_hbm.at[row, pl.ds(col, _CHUNK)], x_v, sem_x)
    cp_p = pltpu.async_copy(pos_hbm.at[pl.ds(col, _CHUNK)], pos_v, sem_p)
    cp_x.wait()
    cp_p.wait()
    for i in range(_CHUNK // _LANES):
        sl = pl.ds(i * _LANES, _LANES)
        x_v[sl] = x_v[sl] + pos_v[sl]
    pltpu.sync_copy(x_v, out_hbm.at[row, pl.ds(col, _CHUNK)])


def kernel(x, pos_table):
    return _pos_add_sc(x, pos_table.reshape(-1))


# speedup vs baseline: 1.0848x; 1.0093x over previous
import functools

import jax
import jax.numpy as jnp
from jax import lax
from jax.experimental import pallas as pl
from jax.experimental.pallas import tpu as pltpu
from jax.experimental.pallas import tpu_sc as plsc

_B = 4
_L = 8192
_NC = 2
_NS = 16
_NW = _NC * _NS
_CHUNK = _B * _L // _NW
_WPR = _L // _CHUNK
_LANES = 16

_mesh = plsc.VectorSubcoreMesh(core_axis_name="c", subcore_axis_name="s")


@functools.partial(
    pl.kernel,
    mesh=_mesh,
    out_type=jax.ShapeDtypeStruct((_B, _L), jnp.float32),
    scratch_types=[
        pltpu.VMEM((_CHUNK,), jnp.float32),
        pltpu.VMEM((_CHUNK,), jnp.float32),
        pltpu.SemaphoreType.DMA,
        pltpu.SemaphoreType.DMA,
    ],
)
def _pos_add_sc(x_hbm, pos_hbm, out_hbm, x_v, pos_v, sem_x, sem_p):
    wid = lax.axis_index("s") * _NC + lax.axis_index("c")
    row = wid // _WPR
    col = (wid % _WPR) * _CHUNK
    cp_x = pltpu.async_copy(x_hbm.at[row, pl.ds(col, _CHUNK)], x_v, sem_x)
    cp_p = pltpu.async_copy(pos_hbm.at[pl.ds(col, _CHUNK)], pos_v, sem_p)
    cp_x.wait()
    cp_p.wait()
    def body(i, _):
        sl = pl.ds(i * _LANES, _LANES)
        x_v[sl] = x_v[sl] + pos_v[sl]
        return _

    lax.fori_loop(0, _CHUNK // _LANES, body, None)
    pltpu.sync_copy(x_v, out_hbm.at[row, pl.ds(col, _CHUNK)])


def kernel(x, pos_table):
    return _pos_add_sc(x, pos_table.reshape(-1))


# speedup vs baseline: 1.1396x; 1.0506x over previous
import functools

import jax
import jax.numpy as jnp
from jax import lax
from jax.experimental import pallas as pl
from jax.experimental.pallas import tpu as pltpu
from jax.experimental.pallas import tpu_sc as plsc

_B = 4
_L = 8192
_NC = 1
_NS = 16
_NW = _NC * _NS
_CHUNK = _B * _L // _NW
_WPR = _L // _CHUNK
_LANES = 16

_mesh = plsc.VectorSubcoreMesh(core_axis_name="c", subcore_axis_name="s",
                               num_cores=1)


@functools.partial(
    pl.kernel,
    mesh=_mesh,
    out_type=jax.ShapeDtypeStruct((_B, _L), jnp.float32),
    scratch_types=[
        pltpu.VMEM((_CHUNK,), jnp.float32),
        pltpu.VMEM((_CHUNK,), jnp.float32),
        pltpu.SemaphoreType.DMA,
        pltpu.SemaphoreType.DMA,
    ],
)
def _pos_add_sc(x_hbm, pos_hbm, out_hbm, x_v, pos_v, sem_x, sem_p):
    wid = lax.axis_index("s") * _NC + lax.axis_index("c")
    row = wid // _WPR
    col = (wid % _WPR) * _CHUNK
    cp_x = pltpu.async_copy(x_hbm.at[row, pl.ds(col, _CHUNK)], x_v, sem_x)
    cp_p = pltpu.async_copy(pos_hbm.at[pl.ds(col, _CHUNK)], pos_v, sem_p)
    cp_x.wait()
    cp_p.wait()
    def body(i, _):
        sl = pl.ds(i * _LANES, _LANES)
        x_v[sl] = x_v[sl] + pos_v[sl]
        return _

    lax.fori_loop(0, _CHUNK // _LANES, body, None)
    pltpu.sync_copy(x_v, out_hbm.at[row, pl.ds(col, _CHUNK)])


def kernel(x, pos_table):
    return _pos_add_sc(x, pos_table.reshape(-1))


# speedup vs baseline: 1.1641x; 1.0215x over previous
import functools

import jax
import jax.numpy as jnp
from jax import lax
from jax.experimental import pallas as pl
from jax.experimental.pallas import tpu as pltpu
from jax.experimental.pallas import tpu_sc as plsc

_B = 4
_L = 8192
_NC = 1
_NS = 16
_NW = _NC * _NS
_CHUNK = _B * _L // _NW
_WPR = _L // _CHUNK
_LANES = 16

_mesh = plsc.VectorSubcoreMesh(core_axis_name="c", subcore_axis_name="s",
                               num_cores=1)


@functools.partial(
    pl.kernel,
    mesh=_mesh,
    out_type=jax.ShapeDtypeStruct((_B, _L), jnp.float32),
    scratch_types=[
        pltpu.VMEM((_CHUNK,), jnp.float32),
        pltpu.VMEM((_CHUNK,), jnp.float32),
        pltpu.SemaphoreType.DMA,
        pltpu.SemaphoreType.DMA,
    ],
)
def _pos_add_sc(x_hbm, pos_hbm, out_hbm, x_v, pos_v, sem_x, sem_p):
    wid = lax.axis_index("s") * _NC + lax.axis_index("c")
    row = wid // _WPR
    col = (wid % _WPR) * _CHUNK
    cp_x = pltpu.async_copy(x_hbm.at[row, pl.ds(col, _CHUNK)], x_v, sem_x)
    cp_p = pltpu.async_copy(pos_hbm.at[pl.ds(col, _CHUNK)], pos_v, sem_p)
    cp_x.wait()
    cp_p.wait()
    _UNROLL = 4

    def body(i, _):
        base = i * (_LANES * _UNROLL)
        for j in range(_UNROLL):
            sl = pl.ds(base + j * _LANES, _LANES)
            x_v[sl] = x_v[sl] + pos_v[sl]
        return _

    lax.fori_loop(0, _CHUNK // (_LANES * _UNROLL), body, None)
    pltpu.sync_copy(x_v, out_hbm.at[row, pl.ds(col, _CHUNK)])


def kernel(x, pos_table):
    return _pos_add_sc(x, pos_table.reshape(-1))


# speedup vs baseline: 1.1710x; 1.0059x over previous
import functools

import jax
import jax.numpy as jnp
from jax import lax
from jax.experimental import pallas as pl
from jax.experimental.pallas import tpu as pltpu
from jax.experimental.pallas import tpu_sc as plsc

_B = 4
_L = 8192
_NC = 1
_NS = 16
_NW = _NC * _NS
_CHUNK = _B * _L // _NW
_WPR = _L // _CHUNK
_LANES = 16

_mesh = plsc.VectorSubcoreMesh(core_axis_name="c", subcore_axis_name="s",
                               num_cores=1)


@functools.partial(
    pl.kernel,
    mesh=_mesh,
    out_type=jax.ShapeDtypeStruct((_B, _L), jnp.float32),
    scratch_types=[
        pltpu.VMEM((_CHUNK,), jnp.float32),
        pltpu.VMEM((_CHUNK,), jnp.float32),
        pltpu.SemaphoreType.DMA,
        pltpu.SemaphoreType.DMA,
    ],
)
def _pos_add_sc(x_hbm, pos_hbm, out_hbm, x_v, pos_v, sem_x, sem_p):
    wid = lax.axis_index("s") * _NC + lax.axis_index("c")
    row = wid // _WPR
    col = (wid % _WPR) * _CHUNK
    cp_x = pltpu.async_copy(x_hbm.at[row, pl.ds(col, _CHUNK)], x_v, sem_x)
    cp_p = pltpu.async_copy(pos_hbm.at[pl.ds(col, _CHUNK)], pos_v, sem_p)
    cp_x.wait()
    cp_p.wait()
    _UNROLL = 8

    def body(i, _):
        base = i * (_LANES * _UNROLL)
        for j in range(_UNROLL):
            sl = pl.ds(base + j * _LANES, _LANES)
            x_v[sl] = x_v[sl] + pos_v[sl]
        return _

    lax.fori_loop(0, _CHUNK // (_LANES * _UNROLL), body, None)
    pltpu.sync_copy(x_v, out_hbm.at[row, pl.ds(col, _CHUNK)])


def kernel(x, pos_table):
    return _pos_add_sc(x, pos_table.reshape(-1))
